# Initial kernel scaffold; baseline (speedup 1.0000x reference)
import jax
import jax.numpy as jnp
from jax.experimental import pallas as pl


def kernel(x, frozen_codebook, W):
    raise NotImplementedError("write your pallas kernel here")


# speedup vs baseline: 1.5814x; 1.5814x over previous
import functools

import jax
import jax.numpy as jnp
from jax import lax
from jax.experimental import pallas as pl
from jax.experimental.pallas import tpu as pltpu
from jax.experimental.pallas import tpu_sc as plsc

B, N, DIM = 16, 576, 256
BN = B * N
K = 8192

KT = 2048
TOK = 256
TOK4 = 1152

NC, NS = 2, 16
NW = NC * NS
BPW = BN // NW
NCH, CH = 3, 96


def _codebook_kernel(cb_ref, cbt_ref, wt_ref, w_ref, rows_ref, cols_ref):
    icb = jnp.dot(cb_ref[...], wt_ref[...], preferred_element_type=jnp.float32)
    rn = jnp.sqrt(jnp.sum(icb * icb, axis=1, keepdims=True))
    rows_ref[...] = icb / jnp.clip(rn, 1e-12)
    icbt = jnp.dot(w_ref[...], cbt_ref[...], preferred_element_type=jnp.float32)
    cn = jnp.sqrt(jnp.sum(icbt * icbt, axis=0, keepdims=True))
    cols_ref[...] = icbt / jnp.clip(cn, 1e-12)


def _assign_kernel(x_ref, cbt_ref, idx_ref):
    xb = x_ref[...]
    nrm = jnp.sqrt(jnp.sum(xb * xb, axis=1, keepdims=True))
    xn = xb / jnp.clip(nrm, 1e-12)
    sim = jnp.dot(xn, cbt_ref[...], preferred_element_type=jnp.float32)
    m = jnp.max(sim, axis=1, keepdims=True)
    ids = lax.broadcasted_iota(jnp.int32, sim.shape, 1)
    cand = jnp.where(sim == m, ids, K)
    idx_ref[...] = jnp.min(cand, axis=1, keepdims=True)


def _rot_kernel(x_ref, q_ref, out_ref, loss_ref):
    i = pl.program_id(0)
    xb = x_ref[...]
    q = q_ref[...]
    nx = jnp.sqrt(jnp.sum(xb * xb, axis=1, keepdims=True))
    xn = xb / jnp.clip(nx, 1e-12)
    ns = jnp.sqrt(jnp.sum(xn * xn, axis=1, keepdims=True))
    nt = jnp.sqrt(jnp.sum(q * q, axis=1, keepdims=True))
    u = xn / jnp.clip(ns, 1e-6)
    qt = q / jnp.clip(nt, 1e-6)
    s = u + qt
    w = s / jnp.clip(jnp.sqrt(jnp.sum(s * s, axis=1, keepdims=True)), 1e-6)
    ew = jnp.sum(xn * w, axis=1, keepdims=True)
    eu = jnp.sum(xn * u, axis=1, keepdims=True)
    rot = xn - 2.0 * ew * w + 2.0 * eu * qt
    out_ref[...] = rot * (nt / jnp.clip(ns, 1e-6))
    d = xn - q
    part = jnp.sum(d * d, axis=(0, 1), keepdims=True) * (1.25 / (BN * DIM))

    @pl.when(i == 0)
    def _():
        loss_ref[...] = jnp.zeros_like(part)

    loss_ref[...] += part


@functools.lru_cache(maxsize=1)
def _make_gather():
    mesh = plsc.VectorSubcoreMesh(
        core_axis_name="c", subcore_axis_name="s",
        num_cores=NC, num_subcores=NS)

    @functools.partial(
        pl.kernel,
        mesh=mesh,
        out_type=jax.ShapeDtypeStruct((NW, NCH, CH, DIM), jnp.float32),
        scratch_types=[
            pltpu.VMEM((NCH, CH), jnp.int32),
            pltpu.VMEM((NCH, CH, DIM), jnp.float32),
            pltpu.SemaphoreType.DMA,
        ],
    )
    def _gather_body(table_hbm, idx_hbm, out_hbm, idx_v, rows_v, sem):
        wid = lax.axis_index("s") * NC + lax.axis_index("c")
        pltpu.sync_copy(idx_hbm.at[wid], idx_v)
        copies = [
            pltpu.async_copy(table_hbm.at[idx_v.at[j]], rows_v.at[j], sem)
            for j in range(NCH)
        ]
        for c in copies:
            c.wait()
        pltpu.sync_copy(rows_v, out_hbm.at[wid])

    return _gather_body


def _gather_kernel(table, idx3):
    return _make_gather()(table, idx3)


def kernel(x, frozen_codebook, W):
    b, n, d = x.shape
    xf = x.reshape(b * n, d)

    rows, cols = pl.pallas_call(
        _codebook_kernel,
        grid=(K // KT,),
        in_specs=[
            pl.BlockSpec((KT, DIM), lambda i: (i, 0)),
            pl.BlockSpec((DIM, KT), lambda i: (0, i)),
            pl.BlockSpec((DIM, DIM), lambda i: (0, 0)),
            pl.BlockSpec((DIM, DIM), lambda i: (0, 0)),
        ],
        out_specs=[
            pl.BlockSpec((KT, DIM), lambda i: (i, 0)),
            pl.BlockSpec((DIM, KT), lambda i: (0, i)),
        ],
        out_shape=[
            jax.ShapeDtypeStruct((K, DIM), jnp.float32),
            jax.ShapeDtypeStruct((DIM, K), jnp.float32),
        ],
    )(frozen_codebook, frozen_codebook.T, W.T, W)

    idx2 = pl.pallas_call(
        _assign_kernel,
        grid=(BN // TOK,),
        in_specs=[
            pl.BlockSpec((TOK, DIM), lambda i: (i, 0)),
            pl.BlockSpec((DIM, K), lambda i: (0, 0)),
        ],
        out_specs=pl.BlockSpec((TOK, 1), lambda i: (i, 0)),
        out_shape=jax.ShapeDtypeStruct((BN, 1), jnp.int32),
    )(xf, cols)
    indices = idx2[:, 0]

    quant = _gather_kernel(rows, indices.reshape(NW, NCH, CH))
    qf = quant.reshape(BN, DIM)

    rot, loss = pl.pallas_call(
        _rot_kernel,
        grid=(BN // TOK4,),
        in_specs=[
            pl.BlockSpec((TOK4, DIM), lambda i: (i, 0)),
            pl.BlockSpec((TOK4, DIM), lambda i: (i, 0)),
        ],
        out_specs=[
            pl.BlockSpec((TOK4, DIM), lambda i: (i, 0)),
            pl.BlockSpec((1, 1), lambda i: (0, 0)),
        ],
        out_shape=[
            jax.ShapeDtypeStruct((BN, DIM), jnp.float32),
            jax.ShapeDtypeStruct((1, 1), jnp.float32),
        ],
    )(xf, qf)

    return rot.reshape(b, n, d), indices.reshape(b, n), loss.reshape(())


# speedup vs baseline: 1.7393x; 1.0998x over previous
import functools

import jax
import jax.numpy as jnp
from jax import lax
from jax.experimental import pallas as pl
from jax.experimental.pallas import tpu as pltpu
from jax.experimental.pallas import tpu_sc as plsc

B, N, DIM = 16, 576, 256
BN = B * N
K = 8192

KT = 2048
NKC = K // KT
GPC = KT // 128
TOK = 256
TOK4 = 1152

NC, NS = 2, 16
NW = NC * NS
BPW = BN // NW
NCH, CH = 3, 96

_MINOR = (((1,), (1,)), ((), ()))


def _codebook_kernel(cb_ref, w_ref, rows_ref, cols_ref):
    cb = cb_ref[...]
    w = w_ref[...]
    icb = lax.dot_general(cb, w, _MINOR, preferred_element_type=jnp.float32)
    rn = jnp.sqrt(jnp.sum(icb * icb, axis=1, keepdims=True))
    rows_ref[...] = icb / jnp.clip(rn, 1e-12)
    icbt = lax.dot_general(w, cb, _MINOR, preferred_element_type=jnp.float32)
    cn = jnp.sqrt(jnp.sum(icbt * icbt, axis=0, keepdims=True))
    cols_ref[0] = icbt / jnp.clip(cn, 1e-12)


def _assign_kernel(x_ref, cbt_ref, idx_ref):
    xb = x_ref[...]
    nrm = jnp.sqrt(jnp.sum(xb * xb, axis=1, keepdims=True))
    xn = xb / jnp.clip(nrm, 1e-12)

    def body(c, carry):
        bv, bg = carry
        sim = jnp.dot(xn, cbt_ref[c], preferred_element_type=jnp.float32)
        for g in range(GPC):
            v = sim[:, g * 128:(g + 1) * 128]
            upd = v > bv
            bv = jnp.where(upd, v, bv)
            bg = jnp.where(upd, c * GPC + g, bg)
        return bv, bg

    bv0 = jnp.full((TOK, 128), -jnp.inf, dtype=jnp.float32)
    bg0 = jnp.zeros((TOK, 128), dtype=jnp.int32)
    bv, bg = lax.fori_loop(0, NKC, body, (bv0, bg0))

    lane = lax.broadcasted_iota(jnp.int32, (TOK, 128), 1)
    gidx = bg * 128 + lane
    m = jnp.max(bv, axis=1, keepdims=True)
    cand = jnp.where(bv == m, gidx, K)
    idx_ref[...] = jnp.min(cand, axis=1, keepdims=True)


def _rot_kernel(x_ref, q_ref, out_ref, loss_ref):
    i = pl.program_id(0)
    xb = x_ref[...]
    q = q_ref[...]
    nx = jnp.sqrt(jnp.sum(xb * xb, axis=1, keepdims=True))
    xn = xb / jnp.clip(nx, 1e-12)
    s = xn + q
    w = s / jnp.clip(jnp.sqrt(jnp.sum(s * s, axis=1, keepdims=True)), 1e-6)
    ew = jnp.sum(xn * w, axis=1, keepdims=True)
    eu = jnp.sum(xn * xn, axis=1, keepdims=True)
    out_ref[...] = xn - 2.0 * ew * w + 2.0 * eu * q
    d = xn - q
    part = jnp.sum(d * d, axis=(0, 1), keepdims=True) * (1.25 / (BN * DIM))

    @pl.when(i == 0)
    def _():
        loss_ref[...] = jnp.zeros_like(part)

    loss_ref[...] += part


@functools.lru_cache(maxsize=1)
def _make_gather():
    mesh = plsc.VectorSubcoreMesh(
        core_axis_name="c", subcore_axis_name="s",
        num_cores=NC, num_subcores=NS)

    @functools.partial(
        pl.kernel,
        mesh=mesh,
        out_type=jax.ShapeDtypeStruct((NW, NCH, CH, DIM), jnp.float32),
        scratch_types=[
            pltpu.VMEM((NCH, CH), jnp.int32),
            pltpu.VMEM((NCH, CH, DIM), jnp.float32),
            pltpu.SemaphoreType.DMA,
        ],
    )
    def _gather_body(table_hbm, idx_hbm, out_hbm, idx_v, rows_v, sem):
        wid = lax.axis_index("s") * NC + lax.axis_index("c")
        pltpu.sync_copy(idx_hbm.at[wid], idx_v)
        copies = [
            pltpu.async_copy(table_hbm.at[idx_v.at[j]], rows_v.at[j], sem)
            for j in range(NCH)
        ]
        for c in copies:
            c.wait()
        pltpu.sync_copy(rows_v, out_hbm.at[wid])

    return _gather_body


def _gather_kernel(table, idx3):
    return _make_gather()(table, idx3)


def kernel(x, frozen_codebook, W):
    b, n, d = x.shape
    xf = x.reshape(b * n, d)

    rows, cols = pl.pallas_call(
        _codebook_kernel,
        grid=(NKC,),
        in_specs=[
            pl.BlockSpec((KT, DIM), lambda i: (i, 0)),
            pl.BlockSpec((DIM, DIM), lambda i: (0, 0)),
        ],
        out_specs=[
            pl.BlockSpec((KT, DIM), lambda i: (i, 0)),
            pl.BlockSpec((1, DIM, KT), lambda i: (i, 0, 0)),
        ],
        out_shape=[
            jax.ShapeDtypeStruct((K, DIM), jnp.float32),
            jax.ShapeDtypeStruct((NKC, DIM, KT), jnp.float32),
        ],
    )(frozen_codebook, W)

    idx2 = pl.pallas_call(
        _assign_kernel,
        grid=(BN // TOK,),
        in_specs=[
            pl.BlockSpec((TOK, DIM), lambda i: (i, 0)),
            pl.BlockSpec((NKC, DIM, KT), lambda i: (0, 0, 0)),
        ],
        out_specs=pl.BlockSpec((TOK, 1), lambda i: (i, 0)),
        out_shape=jax.ShapeDtypeStruct((BN, 1), jnp.int32),
    )(xf, cols)
    indices = idx2[:, 0]

    quant = _gather_kernel(rows, indices.reshape(NW, NCH, CH))
    qf = quant.reshape(BN, DIM)

    rot, loss = pl.pallas_call(
        _rot_kernel,
        grid=(BN // TOK4,),
        in_specs=[
            pl.BlockSpec((TOK4, DIM), lambda i: (i, 0)),
            pl.BlockSpec((TOK4, DIM), lambda i: (i, 0)),
        ],
        out_specs=[
            pl.BlockSpec((TOK4, DIM), lambda i: (i, 0)),
            pl.BlockSpec((1, 1), lambda i: (0, 0)),
        ],
        out_shape=[
            jax.ShapeDtypeStruct((BN, DIM), jnp.float32),
            jax.ShapeDtypeStruct((1, 1), jnp.float32),
        ],
    )(xf, qf)

    return rot.reshape(b, n, d), indices.reshape(b, n), loss.reshape(())


# speedup vs baseline: 2.2574x; 1.2979x over previous
import functools

import jax
import jax.numpy as jnp
from jax import lax
from jax.experimental import pallas as pl
from jax.experimental.pallas import tpu as pltpu
from jax.experimental.pallas import tpu_sc as plsc

B, N, DIM = 16, 576, 256
BN = B * N
K = 8192

KT = 2048
NKC = K // KT
GPC = KT // 128
TOK = 256
TOK4 = 1152

NC, NS = 2, 16
NW = NC * NS
BPW = BN // NW
NCH, CH = 3, 96

_MINOR = (((1,), (1,)), ((), ()))


def _codebook_kernel(cb_ref, w_ref, rows_ref, cols_ref):
    cb = cb_ref[...]
    w = w_ref[...]
    icb = lax.dot_general(cb, w, _MINOR, preferred_element_type=jnp.float32)
    rn = jnp.sqrt(jnp.sum(icb * icb, axis=1, keepdims=True))
    rows_ref[...] = icb / jnp.clip(rn, 1e-12)
    icbt = lax.dot_general(w, cb, _MINOR, preferred_element_type=jnp.float32)
    cn = jnp.sqrt(jnp.sum(icbt * icbt, axis=0, keepdims=True))
    cols_ref[0] = icbt / jnp.clip(cn, 1e-12)


def _assign_kernel(x_ref, cbt_ref, idx_ref):
    xb = x_ref[...]
    nrm = jnp.sqrt(jnp.sum(xb * xb, axis=1, keepdims=True))
    xn = xb / jnp.clip(nrm, 1e-12)

    bv = jnp.full((TOK, 128), -jnp.inf, dtype=jnp.float32)
    bg = jnp.zeros((TOK, 128), dtype=jnp.int32)
    for c in range(NKC):
        sim = jnp.dot(xn, cbt_ref[c], preferred_element_type=jnp.float32)
        for g in range(GPC):
            v = sim[:, g * 128:(g + 1) * 128]
            upd = v > bv
            bv = jnp.where(upd, v, bv)
            bg = jnp.where(upd, c * GPC + g, bg)

    lane = lax.broadcasted_iota(jnp.int32, (TOK, 128), 1)
    gidx = bg * 128 + lane
    m = jnp.max(bv, axis=1, keepdims=True)
    cand = jnp.where(bv == m, gidx, K)
    idx_ref[...] = jnp.min(cand, axis=1, keepdims=True)


def _rot_kernel(x_ref, q_ref, out_ref, loss_ref):
    i = pl.program_id(0)
    xb = x_ref[...]
    q = q_ref[...]
    nx = jnp.sqrt(jnp.sum(xb * xb, axis=1, keepdims=True))
    xn = xb * (1.0 / jnp.clip(nx, 1e-12))
    s = xn + q
    w = s * (1.0 / jnp.clip(jnp.sqrt(jnp.sum(s * s, axis=1, keepdims=True)), 1e-6))
    ew = jnp.sum(xn * w, axis=1, keepdims=True)
    eu = jnp.sum(xn * xn, axis=1, keepdims=True)
    out_ref[...] = xn - 2.0 * ew * w + 2.0 * eu * q
    d = xn - q
    part = jnp.sum(d * d, axis=(0, 1), keepdims=True) * (1.25 / (BN * DIM))

    @pl.when(i == 0)
    def _():
        loss_ref[...] = jnp.zeros_like(part)

    loss_ref[...] += part


@functools.lru_cache(maxsize=1)
def _make_gather():
    mesh = plsc.VectorSubcoreMesh(
        core_axis_name="c", subcore_axis_name="s",
        num_cores=NC, num_subcores=NS)

    @functools.partial(
        pl.kernel,
        mesh=mesh,
        out_type=jax.ShapeDtypeStruct((NW, NCH, CH, DIM), jnp.float32),
        scratch_types=[
            pltpu.VMEM((NCH, CH), jnp.int32),
            pltpu.VMEM((NCH, CH, DIM), jnp.float32),
            pltpu.SemaphoreType.DMA,
        ],
    )
    def _gather_body(table_hbm, idx_hbm, out_hbm, idx_v, rows_v, sem):
        wid = lax.axis_index("s") * NC + lax.axis_index("c")
        pltpu.sync_copy(idx_hbm.at[wid], idx_v)
        copies = [
            pltpu.async_copy(table_hbm.at[idx_v.at[j]], rows_v.at[j], sem)
            for j in range(NCH)
        ]
        for c in copies:
            c.wait()
        pltpu.sync_copy(rows_v, out_hbm.at[wid])

    return _gather_body


def _gather_kernel(table, idx3):
    return _make_gather()(table, idx3)


def kernel(x, frozen_codebook, W):
    b, n, d = x.shape
    xf = x.reshape(b * n, d)

    rows, cols = pl.pallas_call(
        _codebook_kernel,
        grid=(NKC,),
        in_specs=[
            pl.BlockSpec((KT, DIM), lambda i: (i, 0)),
            pl.BlockSpec((DIM, DIM), lambda i: (0, 0)),
        ],
        out_specs=[
            pl.BlockSpec((KT, DIM), lambda i: (i, 0)),
            pl.BlockSpec((1, DIM, KT), lambda i: (i, 0, 0)),
        ],
        out_shape=[
            jax.ShapeDtypeStruct((K, DIM), jnp.float32),
            jax.ShapeDtypeStruct((NKC, DIM, KT), jnp.float32),
        ],
    )(frozen_codebook, W)

    idx2 = pl.pallas_call(
        _assign_kernel,
        grid=(BN // TOK,),
        in_specs=[
            pl.BlockSpec((TOK, DIM), lambda i: (i, 0)),
            pl.BlockSpec((NKC, DIM, KT), lambda i: (0, 0, 0)),
        ],
        out_specs=pl.BlockSpec((TOK, 1), lambda i: (i, 0)),
        out_shape=jax.ShapeDtypeStruct((BN, 1), jnp.int32),
    )(xf, cols)
    indices = idx2[:, 0]

    quant = _gather_kernel(rows, indices.reshape(NW, NCH, CH))
    qf = quant.reshape(BN, DIM)

    rot, loss = pl.pallas_call(
        _rot_kernel,
        grid=(BN // TOK4,),
        in_specs=[
            pl.BlockSpec((TOK4, DIM), lambda i: (i, 0)),
            pl.BlockSpec((TOK4, DIM), lambda i: (i, 0)),
        ],
        out_specs=[
            pl.BlockSpec((TOK4, DIM), lambda i: (i, 0)),
            pl.BlockSpec((1, 1), lambda i: (0, 0)),
        ],
        out_shape=[
            jax.ShapeDtypeStruct((BN, DIM), jnp.float32),
            jax.ShapeDtypeStruct((1, 1), jnp.float32),
        ],
    )(xf, qf)

    return rot.reshape(b, n, d), indices.reshape(b, n), loss.reshape(())
